# baseline (speedup 1.0000x reference)
import functools

import jax
import jax.numpy as jnp
from jax import lax
from jax.experimental import pallas as pl
from jax.experimental.pallas import tpu as pltpu
from jax.experimental.pallas import tpu_sc as plsc

_NUM_CORES = 2
_NUM_SUBCORES = 16
_NW = _NUM_CORES * _NUM_SUBCORES
_CHUNK = 128


def _sc_gather(data, idx3d):
    nw, per_w, chunk = idx3d.shape
    d = data.shape[1]
    mesh = plsc.VectorSubcoreMesh(
        core_axis_name="c", subcore_axis_name="s",
        num_cores=_NUM_CORES, num_subcores=_NUM_SUBCORES)

    @functools.partial(
        pl.kernel,
        out_type=jax.ShapeDtypeStruct((nw * per_w * chunk, d), jnp.float32),
        mesh=mesh,
        compiler_params=pltpu.CompilerParams(use_tc_tiling_on_sc=False),
        scratch_types=[
            pltpu.VMEM((per_w, chunk), jnp.int32),
            pltpu.VMEM((chunk, d), jnp.float32),
            pltpu.SemaphoreType.DMA,
        ],
    )
    def body(data_hbm, idx_hbm, out_hbm, idx_v, rows_v, gsem):
        wid = lax.axis_index("s") * _NUM_CORES + lax.axis_index("c")
        base = wid * per_w
        pltpu.sync_copy(idx_hbm.at[wid], idx_v)

        def step(j, carry):
            pltpu.async_copy(data_hbm.at[idx_v.at[j]], rows_v, gsem).wait()
            pltpu.sync_copy(rows_v, out_hbm.at[pl.ds((base + j) * chunk, chunk)])
            return carry

        lax.fori_loop(0, per_w, step, 0)

    return body(data, idx3d)


def kernel(data, indices, axis):
    v, d = data.shape
    idx = indices.reshape(-1).astype(jnp.int32)
    idx = idx + jnp.asarray(axis, dtype=jnp.int32)
    idx = jnp.where(idx < 0, idx + v, idx)
    per_w = idx.shape[0] // (_NW * _CHUNK)
    idx3d = idx.reshape(_NW, per_w, _CHUNK)
    out = _sc_gather(data, idx3d)
    return out.reshape(indices.shape + (d,))


# speedup vs baseline: 1.0425x; 1.0425x over previous
import functools

import jax
import jax.numpy as jnp
from jax import lax
from jax.experimental import pallas as pl
from jax.experimental.pallas import tpu as pltpu
from jax.experimental.pallas import tpu_sc as plsc

_NUM_CORES = 2
_NUM_SUBCORES = 16
_NW = _NUM_CORES * _NUM_SUBCORES
_CHUNK = 128
_NBUF = 5


def _sc_gather(data, idx3d):
    nw, per_w, chunk = idx3d.shape
    d = data.shape[1]
    mesh = plsc.VectorSubcoreMesh(
        core_axis_name="c", subcore_axis_name="s",
        num_cores=_NUM_CORES, num_subcores=_NUM_SUBCORES)

    nbuf = _NBUF
    assert per_w % nbuf == 0 and per_w > nbuf

    @functools.partial(
        pl.kernel,
        out_type=jax.ShapeDtypeStruct((nw * per_w * chunk, d), jnp.float32),
        mesh=mesh,
        compiler_params=pltpu.CompilerParams(use_tc_tiling_on_sc=False),
        scratch_types=[
            pltpu.VMEM((per_w, chunk), jnp.int32),
            pltpu.VMEM((nbuf, chunk, d), jnp.float32),
            [pltpu.SemaphoreType.DMA] * nbuf,
            [pltpu.SemaphoreType.DMA] * nbuf,
        ],
    )
    def body(data_hbm, idx_hbm, out_hbm, idx_v, rows_v, gsems, ssems):
        wid = lax.axis_index("s") * _NUM_CORES + lax.axis_index("c")
        base = wid * per_w
        pltpu.sync_copy(idx_hbm.at[wid], idx_v)

        def gather_start(j, b):
            pltpu.async_copy(data_hbm.at[idx_v.at[j]], rows_v.at[b], gsems[b])

        def gather_wait(j, b):
            pltpu.make_async_copy(
                data_hbm.at[idx_v.at[j]], rows_v.at[b], gsems[b]).wait()

        def store_start(j, b):
            pltpu.async_copy(
                rows_v.at[b], out_hbm.at[pl.ds((base + j) * chunk, chunk)],
                ssems[b])

        def store_wait(j, b):
            pltpu.make_async_copy(
                rows_v.at[b], out_hbm.at[pl.ds((base + j) * chunk, chunk)],
                ssems[b]).wait()

        for b in range(nbuf):
            gather_start(b, b)

        def outer(g, carry):
            j0 = g * nbuf
            for b in range(nbuf):
                gather_wait(j0 + b, b)
                store_start(j0 + b, b)
            for b in range(nbuf):
                jn = j0 + nbuf + b

                @pl.when(jn < per_w)
                def _():
                    store_wait(j0 + b, b)
                    gather_start(jn, b)

            return carry

        lax.fori_loop(0, per_w // nbuf, outer, 0)
        for b in range(nbuf):
            store_wait(per_w - nbuf + b, b)

    return body(data, idx3d)


def kernel(data, indices, axis):
    v, d = data.shape
    idx = indices.reshape(-1).astype(jnp.int32)
    idx = idx + jnp.asarray(axis, dtype=jnp.int32)
    idx = jnp.where(idx < 0, idx + v, idx)
    per_w = idx.shape[0] // (_NW * _CHUNK)
    idx3d = idx.reshape(_NW, per_w, _CHUNK)
    out = _sc_gather(data, idx3d)
    return out.reshape(indices.shape + (d,))


# speedup vs baseline: 1.0469x; 1.0042x over previous
import functools

import jax
import jax.numpy as jnp
from jax import lax
from jax.experimental import pallas as pl
from jax.experimental.pallas import tpu as pltpu
from jax.experimental.pallas import tpu_sc as plsc

_NUM_CORES = 2
_NUM_SUBCORES = 16
_NW = _NUM_CORES * _NUM_SUBCORES
_CHUNK = 128
_NBUF = 10


def _sc_gather(data, idx3d):
    nw, per_w, chunk = idx3d.shape
    d = data.shape[1]
    mesh = plsc.VectorSubcoreMesh(
        core_axis_name="c", subcore_axis_name="s",
        num_cores=_NUM_CORES, num_subcores=_NUM_SUBCORES)

    nbuf = _NBUF
    assert per_w % nbuf == 0 and per_w > nbuf

    @functools.partial(
        pl.kernel,
        out_type=jax.ShapeDtypeStruct((nw * per_w * chunk, d), jnp.float32),
        mesh=mesh,
        compiler_params=pltpu.CompilerParams(use_tc_tiling_on_sc=False),
        scratch_types=[
            pltpu.VMEM((per_w, chunk), jnp.int32),
            pltpu.VMEM((nbuf, chunk, d), jnp.float32),
            [pltpu.SemaphoreType.DMA] * nbuf,
            [pltpu.SemaphoreType.DMA] * nbuf,
        ],
    )
    def body(data_hbm, idx_hbm, out_hbm, idx_v, rows_v, gsems, ssems):
        wid = lax.axis_index("s") * _NUM_CORES + lax.axis_index("c")
        base = wid * per_w
        pltpu.sync_copy(idx_hbm.at[wid], idx_v)

        def gather_start(j, b):
            pltpu.async_copy(data_hbm.at[idx_v.at[j]], rows_v.at[b], gsems[b])

        def gather_wait(j, b):
            pltpu.make_async_copy(
                data_hbm.at[idx_v.at[j]], rows_v.at[b], gsems[b]).wait()

        def store_start(j, b):
            pltpu.async_copy(
                rows_v.at[b], out_hbm.at[pl.ds((base + j) * chunk, chunk)],
                ssems[b])

        def store_wait(j, b):
            pltpu.make_async_copy(
                rows_v.at[b], out_hbm.at[pl.ds((base + j) * chunk, chunk)],
                ssems[b]).wait()

        for b in range(nbuf):
            gather_start(b, b)

        def outer(g, carry):
            j0 = g * nbuf
            for b in range(nbuf):
                gather_wait(j0 + b, b)
                store_start(j0 + b, b)
            for b in range(nbuf):
                jn = j0 + nbuf + b

                @pl.when(jn < per_w)
                def _():
                    store_wait(j0 + b, b)
                    gather_start(jn, b)

            return carry

        lax.fori_loop(0, per_w // nbuf, outer, 0)
        for b in range(nbuf):
            store_wait(per_w - nbuf + b, b)

    return body(data, idx3d)


def kernel(data, indices, axis):
    v, d = data.shape
    idx = indices.reshape(-1).astype(jnp.int32)
    idx = idx + jnp.asarray(axis, dtype=jnp.int32)
    idx = jnp.where(idx < 0, idx + v, idx)
    per_w = idx.shape[0] // (_NW * _CHUNK)
    idx3d = idx.reshape(_NW, per_w, _CHUNK)
    out = _sc_gather(data, idx3d)
    return out.reshape(indices.shape + (d,))
